# baseline (speedup 1.0000x reference)
import functools

import jax
import jax.numpy as jnp
from jax.experimental import pallas as pl
from jax.experimental.pallas import tpu as pltpu

N_TOK, D_MODEL, N_NODES, N_EXPERTS = 8192, 1024, 16, 64
KDIM = D_MODEL * N_NODES
BLOCK_N = 256
CH_T = 16
CH_L = 1024


def _round_to_bf16_in_f32(x):
    u = jax.lax.bitcast_convert_type(x, jnp.int32)
    rounded = (u + 0x7FFF + ((u >> 16) & 1)) & jnp.int32(-65536)
    return jax.lax.bitcast_convert_type(rounded, jnp.float32)


def _router_body(x_ref, wst_ref, bst_ref, wge_ref, bg_ref,
                 gates_ref, load_ref, p2_ref):
    bias = bst_ref[0, 0]

    def chunk(i, _):
        t = (i // (KDIM // CH_L)) * CH_T
        l = (i % (KDIM // CH_L)) * CH_L
        xr = _round_to_bf16_in_f32(x_ref[pl.ds(t, CH_T), pl.ds(l, CH_L)])
        p = xr * wst_ref[0, pl.ds(l, CH_L)][None, :]
        for k in (1, 2, 4, 8):
            p = p + pltpu.roll(p, CH_L - k, axis=1)
        p2_ref[pl.ds(t, CH_T), pl.ds(l, CH_L)] = (p + bias).astype(jnp.bfloat16)
        return 0

    jax.lax.fori_loop(0, (BLOCK_N // CH_T) * (KDIM // CH_L), chunk, 0)

    logits = (
        jax.lax.dot_general(p2_ref[...], wge_ref[...],
                            (((1,), (0,)), ((), ())),
                            preferred_element_type=jnp.float32)
        + bg_ref[...]
    )
    col = jax.lax.broadcasted_iota(jnp.int32, logits.shape, 1)
    m1 = jnp.max(logits, axis=1, keepdims=True)
    i1 = jnp.min(jnp.where(logits == m1, col, N_EXPERTS), axis=1, keepdims=True)
    masked = jnp.where(col == i1, -jnp.inf, logits)
    m2 = jnp.max(masked, axis=1, keepdims=True)
    i2 = jnp.min(jnp.where(masked == m2, col, N_EXPERTS), axis=1, keepdims=True)
    t = jnp.exp(m2 - m1)
    denom = 1.0 + t
    g1 = 1.0 / denom
    g2 = t / denom
    gates = jnp.where(col == i1, g1, 0.0) + jnp.where(col == i2, g2, 0.0)
    gates_ref[...] = gates
    part = jnp.sum((gates > 0.0).astype(jnp.int32), axis=0, keepdims=True)

    @pl.when(pl.program_id(0) == 0)
    def _init():
        load_ref[...] = part

    @pl.when(pl.program_id(0) != 0)
    def _acc():
        load_ref[...] += part


@functools.partial(jax.jit, static_argnames=("interpret",))
def _run(x2, wst, bst, wge, bg, interpret=False):
    grid = (N_TOK // BLOCK_N,)
    gates, load = pl.pallas_call(
        _router_body,
        grid=grid,
        in_specs=[
            pl.BlockSpec((BLOCK_N, KDIM), lambda i: (i, 0)),
            pl.BlockSpec((1, KDIM), lambda i: (0, 0)),
            pl.BlockSpec((1, 1), lambda i: (0, 0)),
            pl.BlockSpec((KDIM, N_EXPERTS), lambda i: (0, 0)),
            pl.BlockSpec((1, N_EXPERTS), lambda i: (0, 0)),
        ],
        out_specs=[
            pl.BlockSpec((BLOCK_N, N_EXPERTS), lambda i: (i, 0)),
            pl.BlockSpec((1, N_EXPERTS), lambda i: (0, 0)),
        ],
        out_shape=[
            jax.ShapeDtypeStruct((N_TOK, N_EXPERTS), jnp.float32),
            jax.ShapeDtypeStruct((1, N_EXPERTS), jnp.int32),
        ],
        scratch_shapes=[pltpu.VMEM((BLOCK_N, KDIM), jnp.bfloat16)],
        interpret=interpret,
    )(x2, wst, bst, wge, bg)
    return gates, load[0]


def _prep(x_trans, W_start, b_start, W_gate, b_gate):
    x2 = x_trans.reshape(N_TOK, KDIM)
    wsb = jax.lax.reduce_precision(W_start[0], 8, 7)
    wst = jnp.tile(wsb, D_MODEL)[None, :]
    bst = jnp.reshape(b_start[0], (1, 1)).astype(jnp.float32)
    wge = jnp.zeros((D_MODEL, N_NODES, N_EXPERTS), jnp.bfloat16)
    wge = wge.at[:, 0, :].set(W_gate.T.astype(jnp.bfloat16))
    wge = wge.reshape(KDIM, N_EXPERTS)
    return x2, wst, bst, wge, b_gate[None, :].astype(jnp.float32)


def kernel(x_trans, W_start, b_start, W_gate, b_gate, W_noise, b_noise, train):
    return _run(*_prep(x_trans, W_start, b_start, W_gate, b_gate))


# speedup vs baseline: 1.5177x; 1.5177x over previous
import functools

import jax
import jax.numpy as jnp
from jax.experimental import pallas as pl
from jax.experimental.pallas import tpu as pltpu

N_TOK, D_MODEL, N_NODES, N_EXPERTS = 8192, 1024, 16, 64
KDIM = D_MODEL * N_NODES
BLOCK_N = 256
CH_T = 16
CH_L = 1024
UNROLL = 2


def _round_to_bf16_in_f32(x):
    u = jax.lax.bitcast_convert_type(x, jnp.int32)
    rounded = (u + 0x7FFF + ((u >> 16) & 1)) & jnp.int32(-65536)
    return jax.lax.bitcast_convert_type(rounded, jnp.float32)


def _router_body(x_ref, wst_ref, bst_ref, wge_ref, bg_ref,
                 gates_ref, load_ref, p2_ref):
    bias = bst_ref[0, 0]

    n_l = KDIM // CH_L

    def chunk(i, _):
        t = (i // (n_l // UNROLL)) * CH_T
        lbase = (i % (n_l // UNROLL)) * (CH_L * UNROLL)
        for g in range(UNROLL):
            l = lbase + g * CH_L
            xr = _round_to_bf16_in_f32(x_ref[pl.ds(t, CH_T), pl.ds(l, CH_L)])
            p = xr * wst_ref[0, pl.ds(l, CH_L)][None, :]
            for k in (1, 2, 4, 8):
                p = p + pltpu.roll(p, CH_L - k, axis=1)
            p2_ref[pl.ds(t, CH_T), pl.ds(l, CH_L)] = (p + bias).astype(jnp.bfloat16)
        return 0

    jax.lax.fori_loop(0, (BLOCK_N // CH_T) * (n_l // UNROLL), chunk, 0)

    logits = (
        jax.lax.dot_general(p2_ref[...], wge_ref[...],
                            (((1,), (0,)), ((), ())),
                            preferred_element_type=jnp.float32)
        + bg_ref[...]
    )
    col = jax.lax.broadcasted_iota(jnp.int32, logits.shape, 1)
    m1 = jnp.max(logits, axis=1, keepdims=True)
    i1 = jnp.min(jnp.where(logits == m1, col, N_EXPERTS), axis=1, keepdims=True)
    masked = jnp.where(col == i1, -jnp.inf, logits)
    m2 = jnp.max(masked, axis=1, keepdims=True)
    i2 = jnp.min(jnp.where(masked == m2, col, N_EXPERTS), axis=1, keepdims=True)
    t = jnp.exp(m2 - m1)
    denom = 1.0 + t
    g1 = 1.0 / denom
    g2 = t / denom
    gates = jnp.where(col == i1, g1, 0.0) + jnp.where(col == i2, g2, 0.0)
    gates_ref[...] = gates
    part = jnp.sum((gates > 0.0).astype(jnp.int32), axis=0, keepdims=True)

    @pl.when(pl.program_id(0) == 0)
    def _init():
        load_ref[...] = part

    @pl.when(pl.program_id(0) != 0)
    def _acc():
        load_ref[...] += part


@functools.partial(jax.jit, static_argnames=("interpret",))
def _run(x2, wst, bst, wge, bg, interpret=False):
    grid = (N_TOK // BLOCK_N,)
    gates, load = pl.pallas_call(
        _router_body,
        grid=grid,
        in_specs=[
            pl.BlockSpec((BLOCK_N, KDIM), lambda i: (i, 0)),
            pl.BlockSpec((1, KDIM), lambda i: (0, 0)),
            pl.BlockSpec((1, 1), lambda i: (0, 0)),
            pl.BlockSpec((KDIM, N_EXPERTS), lambda i: (0, 0)),
            pl.BlockSpec((1, N_EXPERTS), lambda i: (0, 0)),
        ],
        out_specs=[
            pl.BlockSpec((BLOCK_N, N_EXPERTS), lambda i: (i, 0)),
            pl.BlockSpec((1, N_EXPERTS), lambda i: (0, 0)),
        ],
        out_shape=[
            jax.ShapeDtypeStruct((N_TOK, N_EXPERTS), jnp.float32),
            jax.ShapeDtypeStruct((1, N_EXPERTS), jnp.int32),
        ],
        scratch_shapes=[pltpu.VMEM((BLOCK_N, KDIM), jnp.bfloat16)],
        interpret=interpret,
    )(x2, wst, bst, wge, bg)
    return gates, load[0]


def _prep(x_trans, W_start, b_start, W_gate, b_gate):
    x2 = x_trans.reshape(N_TOK, KDIM)
    wsb = jax.lax.reduce_precision(W_start[0], 8, 7)
    wst = jnp.tile(wsb, D_MODEL)[None, :]
    bst = jnp.reshape(b_start[0], (1, 1)).astype(jnp.float32)
    wge = jnp.zeros((D_MODEL, N_NODES, N_EXPERTS), jnp.bfloat16)
    wge = wge.at[:, 0, :].set(W_gate.T.astype(jnp.bfloat16))
    wge = wge.reshape(KDIM, N_EXPERTS)
    return x2, wst, bst, wge, b_gate[None, :].astype(jnp.float32)


def kernel(x_trans, W_start, b_start, W_gate, b_gate, W_noise, b_noise, train):
    return _run(*_prep(x_trans, W_start, b_start, W_gate, b_gate))


# speedup vs baseline: 3.2207x; 2.1221x over previous
import functools

import jax
import jax.numpy as jnp
from jax.experimental import pallas as pl

N_TOK, D_MODEL, N_NODES, N_EXPERTS = 8192, 1024, 16, 64
KDIM = D_MODEL * N_NODES
BLOCK_N = 128


def _round_to_bf16_in_f32(x):
    u = jax.lax.bitcast_convert_type(x, jnp.int32)
    rounded = (u + 0x7FFF + ((u >> 16) & 1)) & jnp.int32(-65536)
    return jax.lax.bitcast_convert_type(rounded, jnp.float32)


def _router_body(x_ref, wst_ref, bst_ref, wg_ref, bg_ref, gates_ref, load_ref):
    xr = _round_to_bf16_in_f32(x_ref[...])
    p = xr * wst_ref[...]
    pt = jnp.transpose(p)
    x1t = jnp.sum(pt.reshape(D_MODEL, N_NODES, BLOCK_N), axis=1)
    x1t = x1t + bst_ref[0, 0]
    logits_t = (
        jax.lax.dot_general(wg_ref[...], x1t.astype(jnp.bfloat16),
                            (((1,), (0,)), ((), ())),
                            preferred_element_type=jnp.float32)
        + bg_ref[...]
    )
    row = jax.lax.broadcasted_iota(jnp.int32, logits_t.shape, 0)
    m1 = jnp.max(logits_t, axis=0, keepdims=True)
    i1 = jnp.min(jnp.where(logits_t == m1, row, N_EXPERTS), axis=0, keepdims=True)
    masked = jnp.where(row == i1, -jnp.inf, logits_t)
    m2 = jnp.max(masked, axis=0, keepdims=True)
    i2 = jnp.min(jnp.where(masked == m2, row, N_EXPERTS), axis=0, keepdims=True)
    t = jnp.exp(m2 - m1)
    denom = 1.0 + t
    g1 = 1.0 / denom
    g2 = t / denom
    gates_t = jnp.where(row == i1, g1, 0.0) + jnp.where(row == i2, g2, 0.0)
    gates = jnp.transpose(gates_t)
    gates_ref[...] = gates
    part = jnp.sum((gates > 0.0).astype(jnp.int32), axis=0, keepdims=True)

    @pl.when(pl.program_id(0) == 0)
    def _init():
        load_ref[...] = part

    @pl.when(pl.program_id(0) != 0)
    def _acc():
        load_ref[...] += part


@functools.partial(jax.jit, static_argnames=("interpret",))
def _run(x2, wst, bst, wg, bg, interpret=False):
    grid = (N_TOK // BLOCK_N,)
    gates, load = pl.pallas_call(
        _router_body,
        grid=grid,
        in_specs=[
            pl.BlockSpec((BLOCK_N, KDIM), lambda i: (i, 0)),
            pl.BlockSpec((1, KDIM), lambda i: (0, 0)),
            pl.BlockSpec((1, 1), lambda i: (0, 0)),
            pl.BlockSpec((N_EXPERTS, D_MODEL), lambda i: (0, 0)),
            pl.BlockSpec((N_EXPERTS, 1), lambda i: (0, 0)),
        ],
        out_specs=[
            pl.BlockSpec((BLOCK_N, N_EXPERTS), lambda i: (i, 0)),
            pl.BlockSpec((1, N_EXPERTS), lambda i: (0, 0)),
        ],
        out_shape=[
            jax.ShapeDtypeStruct((N_TOK, N_EXPERTS), jnp.float32),
            jax.ShapeDtypeStruct((1, N_EXPERTS), jnp.int32),
        ],
        interpret=interpret,
    )(x2, wst, bst, wg, bg)
    return gates, load[0]


def _prep(x_trans, W_start, b_start, W_gate, b_gate):
    x2 = x_trans.reshape(N_TOK, KDIM)
    wsb = jax.lax.reduce_precision(W_start[0], 8, 7)
    wst = jnp.tile(wsb, D_MODEL)[None, :]
    bst = jnp.reshape(b_start[0], (1, 1)).astype(jnp.float32)
    wg = W_gate.astype(jnp.bfloat16)
    bg = b_gate.astype(jnp.float32)[:, None]
    return x2, wst, bst, wg, bg


def kernel(x_trans, W_start, b_start, W_gate, b_gate, W_noise, b_noise, train):
    return _run(*_prep(x_trans, W_start, b_start, W_gate, b_gate))
